# baseline (speedup 1.0000x reference)
import functools

import jax
import jax.numpy as jnp
from jax import lax
from jax.experimental import pallas as pl
from jax.experimental.pallas import tpu as pltpu
from jax.experimental.pallas import tpu_sc as plsc

V = 1000000
D = 64
B = 16384
NS = 5

NC = 2
NSC = 16
NW = NC * NSC
EPW = B // NW
C = 128
NCHUNK = EPW // C
NR = C * NS

WB = 8192
LOG2WB = 13
NBLK = (V + 2 * WB - 1) // (2 * WB)
VP = NBLK * WB


def _xpose_kernel(x_ref, out_ref):
    x = x_ref[...]
    eye = (lax.broadcasted_iota(jnp.int32, (D, D), 0)
           == lax.broadcasted_iota(jnp.int32, (D, D), 1)).astype(jnp.float32)
    out_ref[:, :D] = x[:, :WB].T
    out_ref[:, D:] = lax.dot_general(
        x[:, WB:], eye, (((0,), (0,)), ((), ())),
        precision=lax.Precision.HIGHEST)


_xpose = pl.pallas_call(
    _xpose_kernel,
    grid=(NBLK,),
    in_specs=[pl.BlockSpec((D, 2 * WB), lambda i: (0, i))],
    out_specs=pl.BlockSpec((WB, 2 * D), lambda i: (i, 0)),
    out_shape=jax.ShapeDtypeStruct((VP, 2 * D), jnp.float32),
)


def _packed_row(v):
    return lax.shift_left(lax.shift_right_logical(v, LOG2WB + 1), LOG2WB) + (v & (WB - 1))


_mesh = plsc.VectorSubcoreMesh(
    core_axis_name="c", subcore_axis_name="s", num_cores=NC, num_subcores=NSC
)


@functools.partial(
    pl.kernel,
    out_type=jax.ShapeDtypeStruct((NW, 8, EPW), jnp.float32),
    mesh=_mesh,
    compiler_params=pltpu.CompilerParams(needs_layout_passes=False),
    scratch_types=[
        pltpu.VMEM((C,), jnp.int32),
        pltpu.VMEM((C,), jnp.int32),
        pltpu.VMEM((NR,), jnp.int32),
        pltpu.VMEM((C,), jnp.int32),
        pltpu.VMEM((C,), jnp.int32),
        pltpu.VMEM((NR,), jnp.int32),
        pltpu.VMEM((C, 128), jnp.float32),
        pltpu.VMEM((C, 128), jnp.float32),
        pltpu.VMEM((NR, 128), jnp.float32),
        pltpu.VMEM((8, C), jnp.float32),
        pltpu.SemaphoreType.DMA,
    ],
)
def _sc_dots(in_w, out_w, noise_w, in_tbl2, out_tbl2, dots_hbm,
             iidx, oidx, nidx, irow, orow, nrow, irows, orows, nrows,
             dots_v, sem):
    wid = lax.axis_index("s") * NC + lax.axis_index("c")
    lane = lax.iota(jnp.int32, 16)
    zeros16 = jnp.zeros((16,), jnp.float32)

    def chunk_body(c_i, carry):
        base = wid * EPW + c_i * C

        pltpu.sync_copy(in_w.at[pl.ds(base, C)], iidx)
        pltpu.sync_copy(out_w.at[pl.ds(base, C)], oidx)
        pltpu.sync_copy(noise_w.at[pl.ds(base * NS, NR)], nidx)

        for t in range(C // 16):
            s = pl.ds(t * 16, 16)
            irow[s] = _packed_row(iidx[s])
            orow[s] = _packed_row(oidx[s])
        for t in range(NR // 16):
            s = pl.ds(t * 16, 16)
            nrow[s] = _packed_row(nidx[s])

        cps = [
            pltpu.async_copy(in_tbl2.at[irow], irows, sem),
            pltpu.async_copy(out_tbl2.at[orow], orows, sem),
        ]
        for n in range(NS):
            cps.append(
                pltpu.async_copy(out_tbl2.at[nrow.at[pl.ds(n * C, C)]],
                                 nrows.at[pl.ds(n * C, C)], sem))
        for cp in cps:
            cp.wait()

        def group_body(g, carry2):
            e_vec = g * 16 + lane
            hin = ((plsc.load_gather(iidx, [e_vec]) >> LOG2WB) & 1) * 64
            hout = ((plsc.load_gather(oidx, [e_vec]) >> LOG2WB) & 1) * 64
            rvecs = []
            hns = []
            for n in range(NS):
                r_vec = e_vec * NS + n
                rvecs.append(r_vec)
                hns.append(((plsc.load_gather(nidx, [r_vec]) >> LOG2WB) & 1) * 64)
            accp = zeros16
            accn = [zeros16] * NS
            for d0 in range(D):
                dv = (d0 + lane) & 63
                a = plsc.load_gather(irows, [e_vec, hin + dv])
                b = plsc.load_gather(orows, [e_vec, hout + dv])
                accp = accp + a * b
                for n in range(NS):
                    cn = plsc.load_gather(nrows, [rvecs[n], hns[n] + dv])
                    accn[n] = accn[n] + a * cn
            s = pl.ds(g * 16, 16)
            dots_v[0, s] = accp
            for n in range(NS):
                dots_v[1 + n, s] = accn[n]
            dots_v[6, s] = zeros16
            dots_v[7, s] = zeros16
            return carry2

        lax.fori_loop(0, C // 16, group_body, 0)

        pltpu.sync_copy(dots_v, dots_hbm.at[wid, :, pl.ds(c_i * C, C)])
        return carry

    lax.fori_loop(0, NCHUNK, chunk_body, 0)


def _tc_loss_kernel(dots_ref, out_ref):
    x = dots_ref[...]
    row = lax.broadcasted_iota(jnp.int32, x.shape, 1)
    t = jnp.where(row == 0, x, -x)
    terms = jnp.log(1.0 / (1.0 + jnp.exp(-t)))
    terms = jnp.where(row < 6, terms, 0.0)
    out_ref[0, 0] = -jnp.sum(terms) / B


_tc_loss = pl.pallas_call(
    _tc_loss_kernel,
    out_shape=jax.ShapeDtypeStruct((1, 1), jnp.float32),
    out_specs=pl.BlockSpec(memory_space=pltpu.SMEM),
)


def kernel(input_words, output_words, noise_words, in_embed_weight, out_embed_weight):
    in_tbl2 = _xpose(in_embed_weight.T)
    out_tbl2 = _xpose(out_embed_weight.T)
    dots = _sc_dots(input_words, output_words, noise_words, in_tbl2, out_tbl2)
    return _tc_loss(dots)[0, 0]


# speedup vs baseline: 1.0378x; 1.0378x over previous
import functools

import jax
import jax.numpy as jnp
from jax import lax
from jax.experimental import pallas as pl
from jax.experimental.pallas import tpu as pltpu
from jax.experimental.pallas import tpu_sc as plsc

V = 1000000
D = 64
B = 16384
NS = 5

NC = 2
NSC = 16
NW = NC * NSC
EPW = B // NW
C = 128
NCHUNK = EPW // C
NR = C * NS

WB = 8192
LOG2WB = 13
NBLK = (V + 2 * WB - 1) // (2 * WB)
VP = NBLK * WB


def _xpose_kernel(x_ref, out_ref):
    x = x_ref[...]
    out_ref[:, :D] = x[:, :WB].T
    out_ref[:, D:] = x[:, WB:].T


_xpose = pl.pallas_call(
    _xpose_kernel,
    grid=(NBLK,),
    in_specs=[pl.BlockSpec((D, 2 * WB), lambda i: (0, i))],
    out_specs=pl.BlockSpec((WB, 2 * D), lambda i: (i, 0)),
    out_shape=jax.ShapeDtypeStruct((VP, 2 * D), jnp.float32),
)


def _packed_row(v):
    return lax.shift_left(lax.shift_right_logical(v, LOG2WB + 1), LOG2WB) + (v & (WB - 1))


_mesh = plsc.VectorSubcoreMesh(
    core_axis_name="c", subcore_axis_name="s", num_cores=NC, num_subcores=NSC
)


@functools.partial(
    pl.kernel,
    out_type=jax.ShapeDtypeStruct((NW, 8, EPW), jnp.float32),
    mesh=_mesh,
    compiler_params=pltpu.CompilerParams(needs_layout_passes=False),
    scratch_types=[
        pltpu.VMEM((C,), jnp.int32),
        pltpu.VMEM((C,), jnp.int32),
        pltpu.VMEM((NR,), jnp.int32),
        pltpu.VMEM((C,), jnp.int32),
        pltpu.VMEM((C,), jnp.int32),
        pltpu.VMEM((NR,), jnp.int32),
        pltpu.VMEM((C, 128), jnp.float32),
        pltpu.VMEM((C, 128), jnp.float32),
        pltpu.VMEM((NR, 128), jnp.float32),
        pltpu.VMEM((8, C), jnp.float32),
        pltpu.SemaphoreType.DMA,
    ],
)
def _sc_dots(in_w, out_w, noise_w, in_tbl2, out_tbl2, dots_hbm,
             iidx, oidx, nidx, irow, orow, nrow, irows, orows, nrows,
             dots_v, sem):
    wid = lax.axis_index("s") * NC + lax.axis_index("c")
    lane = lax.iota(jnp.int32, 16)
    zeros16 = jnp.zeros((16,), jnp.float32)

    def chunk_body(c_i, carry):
        base = wid * EPW + c_i * C

        pltpu.sync_copy(in_w.at[pl.ds(base, C)], iidx)
        pltpu.sync_copy(out_w.at[pl.ds(base, C)], oidx)
        pltpu.sync_copy(noise_w.at[pl.ds(base * NS, NR)], nidx)

        for t in range(C // 16):
            s = pl.ds(t * 16, 16)
            irow[s] = _packed_row(iidx[s])
            orow[s] = _packed_row(oidx[s])
        for t in range(NR // 16):
            s = pl.ds(t * 16, 16)
            nrow[s] = _packed_row(nidx[s])

        cps = [
            pltpu.async_copy(in_tbl2.at[irow], irows, sem),
            pltpu.async_copy(out_tbl2.at[orow], orows, sem),
        ]
        for n in range(NS):
            cps.append(
                pltpu.async_copy(out_tbl2.at[nrow.at[pl.ds(n * C, C)]],
                                 nrows.at[pl.ds(n * C, C)], sem))
        for cp in cps:
            cp.wait()

        def group_body(g, carry2):
            e_vec = g * 16 + lane
            hin = ((plsc.load_gather(iidx, [e_vec]) >> LOG2WB) & 1) * 64
            hout = ((plsc.load_gather(oidx, [e_vec]) >> LOG2WB) & 1) * 64
            rvecs = []
            hns = []
            for n in range(NS):
                r_vec = e_vec * NS + n
                rvecs.append(r_vec)
                hns.append(((plsc.load_gather(nidx, [r_vec]) >> LOG2WB) & 1) * 64)
            accp = zeros16
            accn = [zeros16] * NS
            for d0 in range(D):
                dv = (d0 + lane) & 63
                a = plsc.load_gather(irows, [e_vec, hin + dv])
                b = plsc.load_gather(orows, [e_vec, hout + dv])
                accp = accp + a * b
                for n in range(NS):
                    cn = plsc.load_gather(nrows, [rvecs[n], hns[n] + dv])
                    accn[n] = accn[n] + a * cn
            s = pl.ds(g * 16, 16)
            dots_v[0, s] = accp
            for n in range(NS):
                dots_v[1 + n, s] = accn[n]
            dots_v[6, s] = zeros16
            dots_v[7, s] = zeros16
            return carry2

        lax.fori_loop(0, C // 16, group_body, 0)

        pltpu.sync_copy(dots_v, dots_hbm.at[wid, :, pl.ds(c_i * C, C)])
        return carry

    lax.fori_loop(0, NCHUNK, chunk_body, 0)


def _tc_loss_kernel(dots_ref, out_ref):
    x = dots_ref[...]
    row = lax.broadcasted_iota(jnp.int32, x.shape, 1)
    t = jnp.where(row == 0, x, -x)
    terms = jnp.log(1.0 / (1.0 + jnp.exp(-t)))
    terms = jnp.where(row < 6, terms, 0.0)
    out_ref[0, 0] = -jnp.sum(terms) / B


_tc_loss = pl.pallas_call(
    _tc_loss_kernel,
    out_shape=jax.ShapeDtypeStruct((1, 1), jnp.float32),
    out_specs=pl.BlockSpec(memory_space=pltpu.SMEM),
)


def kernel(input_words, output_words, noise_words, in_embed_weight, out_embed_weight):
    in_tbl2 = _xpose(in_embed_weight.T)
    out_tbl2 = _xpose(out_embed_weight.T)
    dots = _sc_dots(input_words, output_words, noise_words, in_tbl2, out_tbl2)
    return _tc_loss(dots)[0, 0]


# speedup vs baseline: 1.1028x; 1.0626x over previous
import functools

import jax
import jax.numpy as jnp
from jax import lax
from jax.experimental import pallas as pl
from jax.experimental.pallas import tpu as pltpu
from jax.experimental.pallas import tpu_sc as plsc

V = 1000000
D = 64
B = 16384
NS = 5

NC = 2
NSC = 16
NW = NC * NSC
EPW = B // NW
C = 128
NCHUNK = EPW // C
NR = C * NS

WB = 16384
LOG2WB = 14
NBLK = (V + 2 * WB - 1) // (2 * WB)
VP = NBLK * WB


def _xpose_kernel(x_ref, out_ref):
    x = x_ref[...]
    out_ref[:, :D] = x[:, :WB].T
    out_ref[:, D:] = x[:, WB:].T


_xpose = pl.pallas_call(
    _xpose_kernel,
    grid=(NBLK,),
    in_specs=[pl.BlockSpec((D, 2 * WB), lambda i: (0, i))],
    out_specs=pl.BlockSpec((WB, 2 * D), lambda i: (i, 0)),
    out_shape=jax.ShapeDtypeStruct((VP, 2 * D), jnp.float32),
)


def _packed_row(v):
    return lax.shift_left(lax.shift_right_logical(v, LOG2WB + 1), LOG2WB) + (v & (WB - 1))


_mesh = plsc.VectorSubcoreMesh(
    core_axis_name="c", subcore_axis_name="s", num_cores=NC, num_subcores=NSC
)


@functools.partial(
    pl.kernel,
    out_type=jax.ShapeDtypeStruct((NW, 8, EPW), jnp.float32),
    mesh=_mesh,
    compiler_params=pltpu.CompilerParams(needs_layout_passes=False),
    scratch_types=[
        pltpu.VMEM((C,), jnp.int32),
        pltpu.VMEM((C,), jnp.int32),
        pltpu.VMEM((NR,), jnp.int32),
        pltpu.VMEM((C,), jnp.int32),
        pltpu.VMEM((C,), jnp.int32),
        pltpu.VMEM((NR,), jnp.int32),
        pltpu.VMEM((C, 128), jnp.float32),
        pltpu.VMEM((C, 128), jnp.float32),
        pltpu.VMEM((NR, 128), jnp.float32),
        pltpu.VMEM((8, C), jnp.float32),
        pltpu.SemaphoreType.DMA,
    ],
)
def _sc_dots(in_w, out_w, noise_w, in_tbl2, out_tbl2, dots_hbm,
             iidx, oidx, nidx, irow, orow, nrow, irows, orows, nrows,
             dots_v, sem):
    wid = lax.axis_index("s") * NC + lax.axis_index("c")
    lane = lax.iota(jnp.int32, 16)
    zeros16 = jnp.zeros((16,), jnp.float32)

    def chunk_body(c_i, carry):
        base = wid * EPW + c_i * C

        pltpu.sync_copy(in_w.at[pl.ds(base, C)], iidx)
        pltpu.sync_copy(out_w.at[pl.ds(base, C)], oidx)
        pltpu.sync_copy(noise_w.at[pl.ds(base * NS, NR)], nidx)

        for t in range(C // 16):
            s = pl.ds(t * 16, 16)
            irow[s] = _packed_row(iidx[s])
            orow[s] = _packed_row(oidx[s])
        for t in range(NR // 16):
            s = pl.ds(t * 16, 16)
            nrow[s] = _packed_row(nidx[s])

        cps = [
            pltpu.async_copy(in_tbl2.at[irow], irows, sem),
            pltpu.async_copy(out_tbl2.at[orow], orows, sem),
        ]
        for n in range(NS):
            cps.append(
                pltpu.async_copy(out_tbl2.at[nrow.at[pl.ds(n * C, C)]],
                                 nrows.at[pl.ds(n * C, C)], sem))
        for cp in cps:
            cp.wait()

        def group_body(g, carry2):
            e_vec = g * 16 + lane
            hin = ((plsc.load_gather(iidx, [e_vec]) >> LOG2WB) & 1) * 64
            hout = ((plsc.load_gather(oidx, [e_vec]) >> LOG2WB) & 1) * 64
            rvecs = []
            hns = []
            for n in range(NS):
                r_vec = e_vec * NS + n
                rvecs.append(r_vec)
                hns.append(((plsc.load_gather(nidx, [r_vec]) >> LOG2WB) & 1) * 64)
            accp = zeros16
            accn = [zeros16] * NS
            for d0 in range(D):
                dv = (d0 + lane) & 63
                a = plsc.load_gather(irows, [e_vec, hin + dv])
                b = plsc.load_gather(orows, [e_vec, hout + dv])
                accp = accp + a * b
                for n in range(NS):
                    cn = plsc.load_gather(nrows, [rvecs[n], hns[n] + dv])
                    accn[n] = accn[n] + a * cn
            s = pl.ds(g * 16, 16)
            dots_v[0, s] = accp
            for n in range(NS):
                dots_v[1 + n, s] = accn[n]
            dots_v[6, s] = zeros16
            dots_v[7, s] = zeros16
            return carry2

        lax.fori_loop(0, C // 16, group_body, 0)

        pltpu.sync_copy(dots_v, dots_hbm.at[wid, :, pl.ds(c_i * C, C)])
        return carry

    lax.fori_loop(0, NCHUNK, chunk_body, 0)


def _tc_loss_kernel(dots_ref, out_ref):
    x = dots_ref[...]
    row = lax.broadcasted_iota(jnp.int32, x.shape, 1)
    t = jnp.where(row == 0, x, -x)
    terms = jnp.log(1.0 / (1.0 + jnp.exp(-t)))
    terms = jnp.where(row < 6, terms, 0.0)
    out_ref[0, 0] = -jnp.sum(terms) / B


_tc_loss = pl.pallas_call(
    _tc_loss_kernel,
    out_shape=jax.ShapeDtypeStruct((1, 1), jnp.float32),
    out_specs=pl.BlockSpec(memory_space=pltpu.SMEM),
)


def kernel(input_words, output_words, noise_words, in_embed_weight, out_embed_weight):
    in_tbl2 = _xpose(in_embed_weight.T)
    out_tbl2 = _xpose(out_embed_weight.T)
    dots = _sc_dots(input_words, output_words, noise_words, in_tbl2, out_tbl2)
    return _tc_loss(dots)[0, 0]
